# baseline (speedup 1.0000x reference)
import functools

import jax
import jax.numpy as jnp
from jax import lax
from jax.experimental import pallas as pl
from jax.experimental.pallas import tpu as pltpu
from jax.experimental.pallas import tpu_sc as plsc

N, H, W, K = 4, 512, 512, 4
B = N * H * W * K
F = 200000
V = 100000
FP = 212992
D = 8

NC, NS = 2, 16
NW = NC * NS

S1_FACES = 1024
S1_IDX_ROWS = S1_FACES * 3 // 128
S1_ITERS = FP // NS // S1_FACES
S1_VECS = S1_FACES * 3 // 16

S2_SUB = 16
S2_PIX = S2_SUB * 128
PIX_PER_W = B // NW
S2_ITERS = PIX_PER_W // S2_PIX

_mesh = plsc.VectorSubcoreMesh(core_axis_name="c", subcore_axis_name="s")


@functools.partial(
    pl.kernel,
    mesh=_mesh,
    out_type=(jax.ShapeDtypeStruct((B * 3,), jnp.float32),
              jax.ShapeDtypeStruct((FP, D), jnp.float32)),
    scratch_types=[
        pltpu.VMEM((S1_IDX_ROWS, 128), jnp.int32),
        pltpu.VMEM((S1_FACES * 3, D), jnp.float32),
        pltpu.VMEM((S1_FACES, D), jnp.float32),
        pltpu.VMEM((S2_SUB, 128), jnp.int32),
        pltpu.VMEM((S2_PIX, D), jnp.float32),
        pltpu.VMEM((S2_PIX * 3,), jnp.float32),
        pltpu.SemaphoreType.DMA,
    ],
    compiler_params=pltpu.CompilerParams(needs_layout_passes=False,
                                         use_tc_tiling_on_sc=False),
)
def _phong_kernel(p2f2d, faces2d, vn, out, table, s1_idx, s1_rows, s1_out,
                  s2_idx, s2_rows, s2_pack, sem):
    c = lax.axis_index("c")
    s = lax.axis_index("s")
    wid = s * NC + c

    def s1_step(it, carry):
        f0 = s * (FP // NS) + it * S1_FACES
        pltpu.sync_copy(faces2d.at[pl.ds(s * (S1_ITERS * S1_IDX_ROWS)
                                         + it * S1_IDX_ROWS, S1_IDX_ROWS)],
                        s1_idx)
        cps = [pltpu.async_copy(vn.at[s1_idx.at[j]],
                                s1_rows.at[pl.ds(j * 128, 128)], sem)
               for j in range(S1_IDX_ROWS)]
        for cp in cps:
            cp.wait()

        def cvec(t, carry2):
            m = t * 16 + lax.iota(jnp.int32, 16)
            fi = m // 3
            cc = m - fi * 3
            r = m - cc
            g0 = plsc.load_gather(s1_rows, [r, cc])
            g1 = plsc.load_gather(s1_rows, [r + 1, cc])
            g2 = plsc.load_gather(s1_rows, [r + 2, cc])
            plsc.store_scatter(s1_out, [fi, cc], g0 + g1 + g2)
            return carry2

        lax.fori_loop(0, S1_VECS, cvec, 0)
        pltpu.sync_copy(s1_out, table.at[pl.ds(f0, S1_FACES)])
        return carry

    lax.fori_loop(0, S1_ITERS, s1_step, 0)
    plsc.subcore_barrier()

    lane = lax.iota(jnp.int32, 16)
    rowpat = [(16 * j + lane) // 3 for j in range(3)]
    colpat = [(16 * j + lane) - 3 * rowpat[j] for j in range(3)]

    def s2_step(g, carry):
        p0 = wid * PIX_PER_W + g * S2_PIX
        pltpu.sync_copy(p2f2d.at[pl.ds(wid * (S2_ITERS * S2_SUB) + g * S2_SUB,
                                       S2_SUB)], s2_idx)
        cps = [pltpu.async_copy(table.at[s2_idx.at[j]],
                                s2_rows.at[pl.ds(j * 128, 128)], sem)
               for j in range(S2_SUB)]
        for cp in cps:
            cp.wait()

        def pack(u, carry2):
            r0 = 16 * u
            for j in range(3):
                val = plsc.load_gather(s2_rows, [rowpat[j] + r0, colpat[j]])
                s2_pack[pl.ds(48 * u + 16 * j, 16)] = val
            return carry2

        lax.fori_loop(0, S2_PIX // 16, pack, 0)
        pltpu.sync_copy(s2_pack, out.at[pl.ds(3 * p0, S2_PIX * 3)])
        return carry

    lax.fori_loop(0, S2_ITERS, s2_step, 0)


def kernel(pix_to_face, faces, vertex_normals):
    p2f2d = pix_to_face.astype(jnp.int32).reshape(B // 128, 128)
    facesp = jnp.concatenate(
        [faces.astype(jnp.int32),
         jnp.zeros((FP - F, 3), jnp.int32)], axis=0)
    faces2d = facesp.reshape(FP * 3 // 128, 128)
    vn8 = jnp.pad(vertex_normals, ((0, 0), (0, D - 3)))
    out, _ = _phong_kernel(p2f2d, faces2d, vn8)
    return out.reshape(N, H, W, K, 3)


# speedup vs baseline: 4.5016x; 4.5016x over previous
import functools

import jax
import jax.numpy as jnp
from jax import lax
from jax.experimental import pallas as pl
from jax.experimental.pallas import tpu as pltpu
from jax.experimental.pallas import tpu_sc as plsc

N, H, W, K = 4, 512, 512, 4
B = N * H * W * K
F = 200000
V = 100000
FP = 212992
D = 8

NC, NS = 2, 16
NW = NC * NS

S1_FACES = 1024
S1_IDX_ROWS = S1_FACES * 3 // 128
S1_ITERS = FP // NS // S1_FACES
S1_VECS = S1_FACES // 16

NG = N * H
GPW = NG // NW
G_IDX_ROWS = 16
G_OUT_ROWS = 48

_mesh = plsc.VectorSubcoreMesh(core_axis_name="c", subcore_axis_name="s")


@functools.partial(
    pl.kernel,
    mesh=_mesh,
    out_type=(jax.ShapeDtypeStruct((NG * G_OUT_ROWS, 128), jnp.float32),
              jax.ShapeDtypeStruct((FP,), jnp.float32),
              jax.ShapeDtypeStruct((FP,), jnp.float32),
              jax.ShapeDtypeStruct((FP,), jnp.float32)),
    scratch_types=[
        pltpu.VMEM((S1_IDX_ROWS, 128), jnp.int32),
        pltpu.VMEM((S1_FACES * 3, D), jnp.float32),
        pltpu.VMEM((S1_FACES,), jnp.float32),
        pltpu.VMEM((S1_FACES,), jnp.float32),
        pltpu.VMEM((S1_FACES,), jnp.float32),
        pltpu.VMEM((G_IDX_ROWS, 128), jnp.int32),
        pltpu.VMEM((G_OUT_ROWS, 128), jnp.float32),
        pltpu.SemaphoreType.DMA,
    ],
    compiler_params=pltpu.CompilerParams(needs_layout_passes=False,
                                         use_tc_tiling_on_sc=False),
)
def _phong_kernel(p2f_nat, faces2d, vn, out, tx, ty, tz,
                  s1_idx, s1_rows, s1x, s1y, s1z, s2_idx, s2_tile, sem):
    c = lax.axis_index("c")
    s = lax.axis_index("s")
    wid = s * NC + c
    lane = lax.iota(jnp.int32, 16)

    def s1_step(it, carry):
        f0 = s * (FP // NS) + it * S1_FACES
        pltpu.sync_copy(faces2d.at[pl.ds(s * (S1_ITERS * S1_IDX_ROWS)
                                         + it * S1_IDX_ROWS, S1_IDX_ROWS)],
                        s1_idx)
        cps = [pltpu.async_copy(vn.at[s1_idx.at[j]],
                                s1_rows.at[pl.ds(j * 128, 128)], sem)
               for j in range(S1_IDX_ROWS)]
        for cp in cps:
            cp.wait()

        def cvec(u, carry2):
            r0 = 48 * u + 3 * lane
            for comp, buf in ((0, s1x), (1, s1y), (2, s1z)):
                cv = jnp.full((16,), comp, jnp.int32)
                val = (plsc.load_gather(s1_rows, [r0, cv])
                       + plsc.load_gather(s1_rows, [r0 + 1, cv])
                       + plsc.load_gather(s1_rows, [r0 + 2, cv]))
                buf[pl.ds(16 * u, 16)] = val
            return carry2

        lax.fori_loop(0, S1_VECS, cvec, 0)
        pltpu.sync_copy(s1x, tx.at[pl.ds(f0, S1_FACES)])
        pltpu.sync_copy(s1y, ty.at[pl.ds(f0, S1_FACES)])
        pltpu.sync_copy(s1z, tz.at[pl.ds(f0, S1_FACES)])
        return carry

    lax.fori_loop(0, S1_ITERS, s1_step, 0)
    plsc.subcore_barrier()

    def s2_step(g, carry):
        gi = wid * GPW + g
        pltpu.sync_copy(p2f_nat.at[pl.ds(gi * G_IDX_ROWS, G_IDX_ROWS)],
                        s2_idx)
        cps = []
        for comp, tref in ((0, tx), (1, ty), (2, tz)):
            for j in range(G_IDX_ROWS):
                cps.append(pltpu.async_copy(tref.at[s2_idx.at[j]],
                                            s2_tile.at[comp * 16 + j], sem))
        for cp in cps:
            cp.wait()
        pltpu.sync_copy(s2_tile, out.at[pl.ds(gi * G_OUT_ROWS, G_OUT_ROWS)])
        return carry

    lax.fori_loop(0, GPW, s2_step, 0)


def kernel(pix_to_face, faces, vertex_normals):
    p2f5 = pix_to_face.astype(jnp.int32).reshape(N, H, W // 128, 128, K)
    p2f_nat = p2f5.transpose(0, 1, 2, 4, 3).reshape(NG * G_IDX_ROWS, 128)
    facesp = jnp.concatenate(
        [faces.astype(jnp.int32),
         jnp.zeros((FP - F, 3), jnp.int32)], axis=0)
    faces2d = facesp.reshape(FP * 3 // 128, 128)
    vn8 = jnp.pad(vertex_normals, ((0, 0), (0, D - 3)))
    out, _, _, _ = _phong_kernel(p2f_nat, faces2d, vn8)
    o6 = out.reshape(N, H, 3, W // 128, K, 128)
    res = o6.transpose(0, 1, 3, 5, 4, 2).reshape(N, H, W, K, 3)
    return res
